# baseline (speedup 1.0000x reference)
import jax
import jax.numpy as jnp
from jax import lax
from jax.experimental import pallas as pl
from jax.experimental.pallas import tpu as pltpu
from jax.experimental.pallas import tpu_sc as plsc

CODEBOOK = 8192
FEAT = 1024
BATCH = 4096

BR = 2048
BC = 1024
NJ = CODEBOOK // BC


def _argmin_body(emb_ref, z_ref, ind_ref, loss_ref,
                 e2s_ref, zf2s_ref, runv_ref, runi_ref):
    i = pl.program_id(0)
    j = pl.program_id(1)
    zf = jnp.reshape(z_ref[...], (BR, FEAT))

    @pl.when(i == 0)
    def _():
        e2s_ref[pl.ds(j * BC, BC), :] = jnp.sum(
            emb_ref[...] ** 2, axis=1, keepdims=True)

    @pl.when(j == 0)
    def _():
        zf2s_ref[...] = jnp.swapaxes(
            jnp.sum(zf ** 2, axis=1, keepdims=True), 0, 1)

    mT = lax.dot_general(
        emb_ref[...], zf,
        dimension_numbers=(((1,), (1,)), ((), ())),
        preferred_element_type=jnp.float32,
    )
    e2 = e2s_ref[pl.ds(j * BC, BC), :]
    dist = (zf2s_ref[...] - 2.0 * mT) + e2
    bmin = jnp.min(dist, axis=0, keepdims=True)
    sub = lax.broadcasted_iota(jnp.int32, dist.shape, 0)
    bidx = jnp.min(jnp.where(dist == bmin, sub, BC), axis=0, keepdims=True)
    bidx = bidx + j * BC

    @pl.when(j == 0)
    def _():
        runv_ref[...] = bmin
        runi_ref[...] = bidx

    @pl.when(j > 0)
    def _():
        upd = bmin < runv_ref[...]
        runi_ref[...] = jnp.where(upd, bidx, runi_ref[...])
        runv_ref[...] = jnp.where(upd, bmin, runv_ref[...])

    @pl.when(j == NJ - 1)
    def _():
        ind_ref[...] = runi_ref[...]
        s = jnp.reshape(jnp.sum(runv_ref[...]), (1, 1))

        @pl.when(i == 0)
        def _():
            loss_ref[...] = s

        @pl.when(i > 0)
        def _():
            loss_ref[...] = loss_ref[...] + s


def _argmin_dist(z, embedding):
    rows, n, d = z.shape
    return pl.pallas_call(
        _argmin_body,
        grid=(rows // BR, NJ),
        in_specs=[
            pl.BlockSpec((BC, FEAT), lambda i, j: (j, 0)),
            pl.BlockSpec((BR, n, d), lambda i, j: (i, 0, 0)),
        ],
        out_specs=[
            pl.BlockSpec((1, BR), lambda i, j: (0, i)),
            pl.BlockSpec((1, 1), lambda i, j: (0, 0)),
        ],
        out_shape=[
            jax.ShapeDtypeStruct((1, rows), jnp.int32),
            jax.ShapeDtypeStruct((1, 1), jnp.float32),
        ],
        scratch_shapes=[
            pltpu.VMEM((CODEBOOK, 1), jnp.float32),
            pltpu.VMEM((1, BR), jnp.float32),
            pltpu.VMEM((1, BR), jnp.float32),
            pltpu.VMEM((1, BR), jnp.int32),
        ],
        compiler_params=pltpu.CompilerParams(
            dimension_semantics=("arbitrary", "arbitrary"),
        ),
    )(embedding, z)


NW = 32


def _sc_gather(embedding, ind):
    rows = ind.shape[0]
    bpw = rows // NW
    ch = 32
    nch = bpw // ch

    def body(emb_hbm, idx_hbm, out_hbm, idx_v, buf0, buf1, g0, g1, w0, w1):
        wid = lax.axis_index("s") * 2 + lax.axis_index("c")
        base = wid * bpw
        pltpu.sync_copy(idx_hbm.at[pl.ds(base, bpw)], idx_v)
        bufs, gs, ws = (buf0, buf1), (g0, g1), (w0, w1)
        gcp = [None] * nch
        wcp = [None] * nch
        for c in range(2):
            gcp[c] = pltpu.async_copy(
                emb_hbm.at[idx_v.at[pl.ds(c * ch, ch)]], bufs[c], gs[c])
        for c in range(nch):
            b = c % 2
            gcp[c].wait()
            wcp[c] = pltpu.async_copy(
                bufs[b], out_hbm.at[pl.ds(base + c * ch, ch)], ws[b])
            if c + 2 < nch:
                wcp[c].wait()
                gcp[c + 2] = pltpu.async_copy(
                    emb_hbm.at[idx_v.at[pl.ds((c + 2) * ch, ch)]],
                    bufs[b], gs[b])
        wcp[nch - 2].wait()
        wcp[nch - 1].wait()

    mesh = plsc.VectorSubcoreMesh(core_axis_name="c", subcore_axis_name="s")
    return pl.kernel(
        body,
        mesh=mesh,
        out_type=jax.ShapeDtypeStruct((rows, FEAT), jnp.float32),
        scratch_types=[
            pltpu.VMEM((bpw,), jnp.int32),
            pltpu.VMEM((ch, FEAT), jnp.float32),
            pltpu.VMEM((ch, FEAT), jnp.float32),
            pltpu.SemaphoreType.DMA,
            pltpu.SemaphoreType.DMA,
            pltpu.SemaphoreType.DMA,
            pltpu.SemaphoreType.DMA,
        ],
    )(embedding, ind)


def kernel(z, embedding):
    Bb, N, D = z.shape
    indr, losssum = _argmin_dist(z, embedding)
    z_q = _sc_gather(embedding, indr.reshape(Bb))
    latent_loss = losssum[0, 0] * (12.5 / (Bb * N * D))
    return (z_q, latent_loss)


# speedup vs baseline: 1.4174x; 1.4174x over previous
import jax
import jax.numpy as jnp
from jax import lax
from jax.experimental import pallas as pl
from jax.experimental.pallas import tpu as pltpu
from jax.experimental.pallas import tpu_sc as plsc

CODEBOOK = 8192
FEAT = 1024
BATCH = 4096

BR = 2048
BC = 1024
NJ = CODEBOOK // BC


def _argmin_body(emb_ref, zf_ref, ind_ref, loss_ref,
                 e2s_ref, zf2s_ref, runv_ref, runi_ref):
    i = pl.program_id(0)
    j = pl.program_id(1)
    zf = zf_ref[...]

    @pl.when(i == 0)
    def _():
        e2s_ref[pl.ds(j * BC, BC), :] = jnp.sum(
            emb_ref[...] ** 2, axis=1, keepdims=True)

    @pl.when(j == 0)
    def _():
        zf2s_ref[...] = jnp.swapaxes(
            jnp.sum(zf ** 2, axis=1, keepdims=True), 0, 1)

    mT = lax.dot_general(
        emb_ref[...], zf,
        dimension_numbers=(((1,), (1,)), ((), ())),
        preferred_element_type=jnp.float32,
    )
    e2 = e2s_ref[pl.ds(j * BC, BC), :]
    dist = (zf2s_ref[...] - 2.0 * mT) + e2
    bmin = jnp.min(dist, axis=0, keepdims=True)
    sub = lax.broadcasted_iota(jnp.int32, dist.shape, 0)
    bidx = jnp.min(jnp.where(dist == bmin, sub, BC), axis=0, keepdims=True)
    bidx = bidx + j * BC

    @pl.when(j == 0)
    def _():
        runv_ref[...] = bmin
        runi_ref[...] = bidx

    @pl.when(j > 0)
    def _():
        upd = bmin < runv_ref[...]
        runi_ref[...] = jnp.where(upd, bidx, runi_ref[...])
        runv_ref[...] = jnp.where(upd, bmin, runv_ref[...])

    @pl.when(j == NJ - 1)
    def _():
        ind_ref[...] = runi_ref[...]
        s = jnp.reshape(jnp.sum(runv_ref[...]), (1, 1))

        @pl.when(i == 0)
        def _():
            loss_ref[...] = s

        @pl.when(i > 0)
        def _():
            loss_ref[...] = loss_ref[...] + s


def _argmin_dist(zf, embedding):
    rows = zf.shape[0]
    return pl.pallas_call(
        _argmin_body,
        grid=(rows // BR, NJ),
        in_specs=[
            pl.BlockSpec((BC, FEAT), lambda i, j: (j, 0)),
            pl.BlockSpec((BR, FEAT), lambda i, j: (i, 0)),
        ],
        out_specs=[
            pl.BlockSpec((1, BR), lambda i, j: (0, i)),
            pl.BlockSpec((1, 1), lambda i, j: (0, 0)),
        ],
        out_shape=[
            jax.ShapeDtypeStruct((1, rows), jnp.int32),
            jax.ShapeDtypeStruct((1, 1), jnp.float32),
        ],
        scratch_shapes=[
            pltpu.VMEM((CODEBOOK, 1), jnp.float32),
            pltpu.VMEM((1, BR), jnp.float32),
            pltpu.VMEM((1, BR), jnp.float32),
            pltpu.VMEM((1, BR), jnp.int32),
        ],
        compiler_params=pltpu.CompilerParams(
            dimension_semantics=("arbitrary", "arbitrary"),
        ),
    )(embedding, zf)


NW = 32


def _sc_gather(embedding, ind):
    rows = ind.shape[0]
    bpw = rows // NW
    ch = 32
    nch = bpw // ch

    def body(emb_hbm, idx_hbm, out_hbm, idx_v, buf0, buf1, g0, g1, w0, w1):
        wid = lax.axis_index("s") * 2 + lax.axis_index("c")
        base = wid * bpw
        pltpu.sync_copy(idx_hbm.at[pl.ds(base, bpw)], idx_v)
        bufs, gs, ws = (buf0, buf1), (g0, g1), (w0, w1)
        gcp = [None] * nch
        wcp = [None] * nch
        for c in range(2):
            gcp[c] = pltpu.async_copy(
                emb_hbm.at[idx_v.at[pl.ds(c * ch, ch)]], bufs[c], gs[c])
        for c in range(nch):
            b = c % 2
            gcp[c].wait()
            wcp[c] = pltpu.async_copy(
                bufs[b], out_hbm.at[pl.ds(base + c * ch, ch)], ws[b])
            if c + 2 < nch:
                wcp[c].wait()
                gcp[c + 2] = pltpu.async_copy(
                    emb_hbm.at[idx_v.at[pl.ds((c + 2) * ch, ch)]],
                    bufs[b], gs[b])
        wcp[nch - 2].wait()
        wcp[nch - 1].wait()

    mesh = plsc.VectorSubcoreMesh(core_axis_name="c", subcore_axis_name="s")
    return pl.kernel(
        body,
        mesh=mesh,
        out_type=jax.ShapeDtypeStruct((rows, FEAT), jnp.float32),
        scratch_types=[
            pltpu.VMEM((bpw,), jnp.int32),
            pltpu.VMEM((ch, FEAT), jnp.float32),
            pltpu.VMEM((ch, FEAT), jnp.float32),
            pltpu.SemaphoreType.DMA,
            pltpu.SemaphoreType.DMA,
            pltpu.SemaphoreType.DMA,
            pltpu.SemaphoreType.DMA,
        ],
    )(embedding, ind)


def kernel(z, embedding):
    Bb, N, D = z.shape
    zf = z.reshape(Bb, N * D)
    indr, losssum = _argmin_dist(zf, embedding)
    z_q = _sc_gather(embedding, indr.reshape(Bb))
    latent_loss = losssum[0, 0] * (12.5 / (Bb * N * D))
    return (z_q, latent_loss)


# speedup vs baseline: 1.5047x; 1.0616x over previous
import jax
import jax.numpy as jnp
from jax import lax
from jax.experimental import pallas as pl
from jax.experimental.pallas import tpu as pltpu
from jax.experimental.pallas import tpu_sc as plsc

CODEBOOK = 8192
FEAT = 1024
BATCH = 4096

BR = 2048
BC = 1024
NJ = CODEBOOK // BC


def _argmin_body(emb_ref, zf_ref, ind_ref, loss_ref,
                 e2s_ref, zf2s_ref, runv_ref, runi_ref):
    i = pl.program_id(0)
    j = pl.program_id(1)

    @pl.when(i == 0)
    def _():
        e2s_ref[pl.ds(j * BC, BC), :] = jnp.sum(
            emb_ref[...] ** 2, axis=1, keepdims=True)

    @pl.when(j == 0)
    def _():
        zf2s_ref[...] = jnp.swapaxes(
            jnp.sum(zf_ref[...] ** 2, axis=1, keepdims=True), 0, 1)

    mT = lax.dot_general(
        emb_ref[...], zf_ref[...],
        dimension_numbers=(((1,), (1,)), ((), ())),
        preferred_element_type=jnp.float32,
    )
    e2 = e2s_ref[pl.ds(j * BC, BC), :]
    dist = (zf2s_ref[...] - 2.0 * mT) + e2
    bmin = jnp.min(dist, axis=0, keepdims=True)
    sub = lax.broadcasted_iota(jnp.int32, dist.shape, 0)
    bidx = jnp.min(jnp.where(dist == bmin, sub, BC), axis=0, keepdims=True)
    bidx = bidx + j * BC

    @pl.when(j == 0)
    def _():
        runv_ref[...] = bmin
        runi_ref[...] = bidx

    @pl.when(j > 0)
    def _():
        upd = bmin < runv_ref[...]
        runi_ref[...] = jnp.where(upd, bidx, runi_ref[...])
        runv_ref[...] = jnp.where(upd, bmin, runv_ref[...])

    @pl.when(j == NJ - 1)
    def _():
        ind_ref[...] = runi_ref[...]
        s = jnp.reshape(jnp.sum(runv_ref[...]), (1, 1))

        @pl.when(i == 0)
        def _():
            loss_ref[...] = s

        @pl.when(i > 0)
        def _():
            loss_ref[...] = loss_ref[...] + s


def _argmin_dist(zf, embedding):
    rows = zf.shape[0]
    return pl.pallas_call(
        _argmin_body,
        grid=(rows // BR, NJ),
        in_specs=[
            pl.BlockSpec((BC, FEAT), lambda i, j: (j, 0)),
            pl.BlockSpec((BR, FEAT), lambda i, j: (i, 0)),
        ],
        out_specs=[
            pl.BlockSpec((1, BR), lambda i, j: (0, i)),
            pl.BlockSpec((1, 1), lambda i, j: (0, 0)),
        ],
        out_shape=[
            jax.ShapeDtypeStruct((1, rows), jnp.int32),
            jax.ShapeDtypeStruct((1, 1), jnp.float32),
        ],
        scratch_shapes=[
            pltpu.VMEM((CODEBOOK, 1), jnp.float32),
            pltpu.VMEM((1, BR), jnp.float32),
            pltpu.VMEM((1, BR), jnp.float32),
            pltpu.VMEM((1, BR), jnp.int32),
        ],
        compiler_params=pltpu.CompilerParams(
            dimension_semantics=("arbitrary", "arbitrary"),
        ),
    )(embedding, zf)


NW = 32


def _sc_gather(embedding, ind):
    rows = ind.shape[0]
    bpw = rows // NW
    ch = 32
    nch = bpw // ch

    def body(emb_hbm, idx_hbm, out_hbm, idx_v, buf0, buf1, g0, g1, w0, w1):
        wid = lax.axis_index("s") * 2 + lax.axis_index("c")
        base = wid * bpw
        pltpu.sync_copy(idx_hbm.at[pl.ds(base, bpw)], idx_v)
        bufs, gs, ws = (buf0, buf1), (g0, g1), (w0, w1)
        gcp = [None] * nch
        wcp = [None] * nch
        for c in range(2):
            gcp[c] = pltpu.async_copy(
                emb_hbm.at[idx_v.at[pl.ds(c * ch, ch)]], bufs[c], gs[c])
        for c in range(nch):
            b = c % 2
            gcp[c].wait()
            wcp[c] = pltpu.async_copy(
                bufs[b], out_hbm.at[pl.ds(base + c * ch, ch)], ws[b])
            if c + 2 < nch:
                wcp[c].wait()
                gcp[c + 2] = pltpu.async_copy(
                    emb_hbm.at[idx_v.at[pl.ds((c + 2) * ch, ch)]],
                    bufs[b], gs[b])
        wcp[nch - 2].wait()
        wcp[nch - 1].wait()

    mesh = plsc.VectorSubcoreMesh(core_axis_name="c", subcore_axis_name="s")
    return pl.kernel(
        body,
        mesh=mesh,
        out_type=jax.ShapeDtypeStruct((rows, FEAT), jnp.float32),
        scratch_types=[
            pltpu.VMEM((bpw,), jnp.int32),
            pltpu.VMEM((ch, FEAT), jnp.float32),
            pltpu.VMEM((ch, FEAT), jnp.float32),
            pltpu.SemaphoreType.DMA,
            pltpu.SemaphoreType.DMA,
            pltpu.SemaphoreType.DMA,
            pltpu.SemaphoreType.DMA,
        ],
    )(embedding, ind)


def kernel(z, embedding):
    Bb, N, D = z.shape
    zf = z.reshape(Bb, N * D)
    indr, losssum = _argmin_dist(zf, embedding)
    z_q = _sc_gather(embedding, indr.reshape(Bb))
    latent_loss = losssum[0, 0] * (12.5 / (Bb * N * D))
    return (z_q, latent_loss)


# speedup vs baseline: 1.5869x; 1.0546x over previous
import jax
import jax.numpy as jnp
from jax import lax
from jax.experimental import pallas as pl
from jax.experimental.pallas import tpu as pltpu
from jax.experimental.pallas import tpu_sc as plsc

CODEBOOK = 8192
FEAT = 1024
BATCH = 4096

BR = 2048
BC = 1024
NJ = CODEBOOK // BC


def _argmin_body(emb_ref, zf_ref, ind_ref, loss_ref,
                 e2s_ref, zf2s_ref, runv_ref, runi_ref):
    i = pl.program_id(0)
    j = pl.program_id(1)

    @pl.when(i == 0)
    def _():
        e2s_ref[pl.ds(j * BC, BC), :] = jnp.sum(
            emb_ref[...] ** 2, axis=1, keepdims=True)

    @pl.when(j == 0)
    def _():
        zf2s_ref[...] = jnp.swapaxes(
            jnp.sum(zf_ref[...] ** 2, axis=1, keepdims=True), 0, 1)

    mT = lax.dot_general(
        emb_ref[...], zf_ref[...],
        dimension_numbers=(((1,), (1,)), ((), ())),
        preferred_element_type=jnp.float32,
    )
    e2 = e2s_ref[pl.ds(j * BC, BC), :]
    dist = (zf2s_ref[...] - 2.0 * mT) + e2
    bmin = jnp.min(dist, axis=0, keepdims=True)
    bidx = jnp.argmin(dist, axis=0).astype(jnp.int32)[None, :]
    bidx = bidx + j * BC

    @pl.when(j == 0)
    def _():
        runv_ref[...] = bmin
        runi_ref[...] = bidx

    @pl.when(j > 0)
    def _():
        upd = bmin < runv_ref[...]
        runi_ref[...] = jnp.where(upd, bidx, runi_ref[...])
        runv_ref[...] = jnp.where(upd, bmin, runv_ref[...])

    @pl.when(j == NJ - 1)
    def _():
        ind_ref[...] = runi_ref[...]
        s = jnp.reshape(jnp.sum(runv_ref[...]), (1, 1))

        @pl.when(i == 0)
        def _():
            loss_ref[...] = s

        @pl.when(i > 0)
        def _():
            loss_ref[...] = loss_ref[...] + s


def _argmin_dist(zf, embedding):
    rows = zf.shape[0]
    return pl.pallas_call(
        _argmin_body,
        grid=(rows // BR, NJ),
        in_specs=[
            pl.BlockSpec((BC, FEAT), lambda i, j: (j, 0)),
            pl.BlockSpec((BR, FEAT), lambda i, j: (i, 0)),
        ],
        out_specs=[
            pl.BlockSpec((1, BR), lambda i, j: (0, i)),
            pl.BlockSpec((1, 1), lambda i, j: (0, 0)),
        ],
        out_shape=[
            jax.ShapeDtypeStruct((1, rows), jnp.int32),
            jax.ShapeDtypeStruct((1, 1), jnp.float32),
        ],
        scratch_shapes=[
            pltpu.VMEM((CODEBOOK, 1), jnp.float32),
            pltpu.VMEM((1, BR), jnp.float32),
            pltpu.VMEM((1, BR), jnp.float32),
            pltpu.VMEM((1, BR), jnp.int32),
        ],
        compiler_params=pltpu.CompilerParams(
            dimension_semantics=("arbitrary", "arbitrary"),
        ),
    )(embedding, zf)


NW = 32


def _sc_gather(embedding, ind):
    rows = ind.shape[0]
    bpw = rows // NW
    ch = 32
    nch = bpw // ch

    def body(emb_hbm, idx_hbm, out_hbm, idx_v, buf0, buf1, g0, g1, w0, w1):
        wid = lax.axis_index("s") * 2 + lax.axis_index("c")
        base = wid * bpw
        pltpu.sync_copy(idx_hbm.at[pl.ds(base, bpw)], idx_v)
        bufs, gs, ws = (buf0, buf1), (g0, g1), (w0, w1)
        gcp = [None] * nch
        wcp = [None] * nch
        for c in range(2):
            gcp[c] = pltpu.async_copy(
                emb_hbm.at[idx_v.at[pl.ds(c * ch, ch)]], bufs[c], gs[c])
        for c in range(nch):
            b = c % 2
            gcp[c].wait()
            wcp[c] = pltpu.async_copy(
                bufs[b], out_hbm.at[pl.ds(base + c * ch, ch)], ws[b])
            if c + 2 < nch:
                wcp[c].wait()
                gcp[c + 2] = pltpu.async_copy(
                    emb_hbm.at[idx_v.at[pl.ds((c + 2) * ch, ch)]],
                    bufs[b], gs[b])
        wcp[nch - 2].wait()
        wcp[nch - 1].wait()

    mesh = plsc.VectorSubcoreMesh(core_axis_name="c", subcore_axis_name="s")
    return pl.kernel(
        body,
        mesh=mesh,
        out_type=jax.ShapeDtypeStruct((rows, FEAT), jnp.float32),
        scratch_types=[
            pltpu.VMEM((bpw,), jnp.int32),
            pltpu.VMEM((ch, FEAT), jnp.float32),
            pltpu.VMEM((ch, FEAT), jnp.float32),
            pltpu.SemaphoreType.DMA,
            pltpu.SemaphoreType.DMA,
            pltpu.SemaphoreType.DMA,
            pltpu.SemaphoreType.DMA,
        ],
    )(embedding, ind)


def kernel(z, embedding):
    Bb, N, D = z.shape
    zf = z.reshape(Bb, N * D)
    indr, losssum = _argmin_dist(zf, embedding)
    z_q = _sc_gather(embedding, indr.reshape(Bb))
    latent_loss = losssum[0, 0] * (12.5 / (Bb * N * D))
    return (z_q, latent_loss)
